# baseline (speedup 1.0000x reference)
import functools

import jax
import jax.numpy as jnp
from jax import lax
from jax.experimental import pallas as pl
from jax.experimental.pallas import tpu as pltpu
from jax.experimental.pallas import tpu_sc as plsc

N0, N1, N2 = 540672, 33792, 3072
D = 128

NC, NS, L = 2, 16, 16
SUB = 128
SUPER = 4096
NSUB = SUPER // SUB

ZR = 8


def _sc_segment_sum(table, src, dst, n_seg, W):
    K = n_seg // W
    P = K // NC
    E = src.shape[0]
    Epad = E + SUPER
    src = jnp.pad(src.astype(jnp.int32), (0, Epad - E))
    dst = jnp.pad(dst.astype(jnp.int32), (0, Epad - E))
    b = jnp.arange(K + 1, dtype=jnp.int32) * W
    bounds = jnp.searchsorted(dst[:E], b, side="left").astype(jnp.int32)
    bounds = jnp.pad(bounds, (0, 32 - (K + 1)), mode="edge")

    R = W // NS
    CW = W // 8
    assert R % ZR == 0 and R % 8 == 0 and W % 8 == 0 and K + 1 <= 32
    assert CW % 8 == 0

    def body(table, src, dst, bounds, sums_hbm, cnt_hbm,
             sums, sidx, draw, didx, rows, cloc, zrow, bvec,
             semg, sems, semz):
        c = lax.axis_index("c")
        s = lax.axis_index("s")
        lanes = lax.iota(jnp.int32, L)
        one16 = jnp.ones((L,), jnp.float32)
        zero16 = jnp.zeros((L,), jnp.float32)

        for j in range(ZR):
            for qq in range(D // L):
                zrow[j, pl.ds(qq * L, L)] = zero16

        pltpu.sync_copy(bounds, bvec)
        bv0 = bvec[pl.ds(0, 16)]
        bv1 = bvec[pl.ds(16, 16)]

        def bnd(i):
            return bv0[i] if i < 16 else bv1[i - 16]

        for p in range(P):
            k = c * P + p
            lo = jnp.where(c == 0, bnd(p), bnd(P + p))
            hi = jnp.where(c == 0, bnd(p + 1), bnd(P + p + 1))
            segbase = k * W

            zcps = []
            for z in range(R // ZR):
                zcps.append(pltpu.async_copy(
                    zrow, sums.at[pl.ds(s * R + z * ZR, ZR)], semz))

            def zloc(r, carry2):
                for qq in range(D // L):
                    cloc[r, pl.ds(qq * L, L)] = zero16
                return carry2

            lax.fori_loop(0, CW + 8, zloc, 0)

            @pl.when(s == 0)
            def _():
                pltpu.async_copy(zrow.at[pl.ds(0, 8)],
                                 sums.at[pl.ds(W, 8)], semz).wait()

            for cp in zcps:
                cp.wait()
            plsc.subcore_barrier()

            q = (hi - lo + NS - 1) // NS
            t_lo = jnp.minimum(lo + s * q, hi)
            t_hi = jnp.minimum(t_lo + q, hi)
            s8 = t_lo - lax.rem(t_lo, 8)
            nsc = (t_hi - s8 + SUPER - 1) // SUPER

            def fire_gather(j):
                return pltpu.async_copy(
                    table.at[sidx.at[pl.ds(j * SUB, SUB)]],
                    rows.at[lax.rem(j, 2)], semg)

            def fire_scatters(j):
                pltpu.async_copy(rows.at[lax.rem(j, 2)],
                                 sums.at[didx.at[j]], sems, add=True)

            def wait_gather(j):
                pltpu.make_async_copy(
                    table.at[sidx.at[pl.ds(j * SUB, SUB)]],
                    rows.at[lax.rem(j, 2)], semg).wait()

            def wait_scatters(j):
                pltpu.make_async_copy(rows.at[lax.rem(j, 2)],
                                      sums.at[didx.at[j]], sems).wait()

            def superchunk(sch, carry):
                sc8 = pl.multiple_of(s8 + sch * SUPER, 8)
                pltpu.sync_copy(src.at[pl.ds(sc8, SUPER)], sidx)
                pltpu.sync_copy(dst.at[pl.ds(sc8, SUPER)], draw)
                rem = t_hi - sc8
                m = jnp.minimum((rem + SUB - 1) // SUB, NSUB)
                fire_gather(0)

                def prep(i, carry2):
                    dv = draw[pl.ds(i * L, L)]
                    g = sc8 + i * L + lanes
                    valid = (g >= t_lo) & (g < t_hi)
                    lidx = jnp.where(valid, dv - segbase, W)
                    didx[i // 8, pl.ds(lax.rem(i, 8) * L, L)] = lidx
                    plsc.addupdate_scatter(
                        cloc,
                        [lax.shift_right_logical(lidx, 3),
                         lax.shift_left(jnp.bitwise_and(lidx, 7), 4) + lanes],
                        one16)
                    return carry2

                lax.fori_loop(0, SUPER // L, prep, 0)

                def inner(j, carry2):
                    @pl.when(j >= 1)
                    def _():
                        wait_scatters(j - 1)

                    @pl.when(j + 1 < m)
                    def _():
                        fire_gather(j + 1)

                    wait_gather(j)
                    fire_scatters(j)
                    return carry2

                lax.fori_loop(0, m, inner, 0)
                wait_scatters(m - 1)
                return carry

            lax.fori_loop(0, nsc, superchunk, 0)
            plsc.subcore_barrier()

            cpo = [pltpu.async_copy(sums.at[pl.ds(s * R, R)],
                                    sums_hbm.at[pl.ds(segbase + s * R, R)],
                                    semz),
                   pltpu.async_copy(cloc.at[pl.ds(0, CW)],
                                    cnt_hbm.at[s, pl.ds(k * CW, CW)],
                                    semz)]
            for cp in cpo:
                cp.wait()
            plsc.subcore_barrier()

    mesh = plsc.VectorSubcoreMesh(core_axis_name="c", subcore_axis_name="s",
                                  num_cores=NC, num_subcores=NS)
    f = pl.kernel(
        body,
        out_type=(jax.ShapeDtypeStruct((n_seg, D), jnp.float32),
                  jax.ShapeDtypeStruct((NS, n_seg // 8, D), jnp.float32)),
        mesh=mesh,
        compiler_params=pltpu.CompilerParams(needs_layout_passes=False),
        scratch_types=[
            pltpu.VMEM_SHARED((W + 8, D), jnp.float32),
            pltpu.VMEM((SUPER,), jnp.int32),
            pltpu.VMEM((SUPER,), jnp.int32),
            pltpu.VMEM((NSUB, SUB), jnp.int32),
            pltpu.VMEM((2, SUB, D), jnp.float32),
            pltpu.VMEM((CW + 8, D), jnp.float32),
            pltpu.VMEM((ZR, D), jnp.float32),
            pltpu.VMEM((32,), jnp.int32),
            pltpu.SemaphoreType.DMA,
            pltpu.SemaphoreType.DMA,
            pltpu.SemaphoreType.DMA,
        ],
    )
    return f(table, src, dst, bounds)


def _tc_linear(sums, cnt, x_tgt, W_l, b_l, W_r, relu):
    N = sums.shape[0]
    B = 1024
    b_l = b_l.reshape(1, D)

    def body(sum_ref, cnt_ref, xt_ref, wl_ref, b_ref, wr_ref, o_ref):
        c2 = jnp.sum(cnt_ref[...], axis=0)
        c3 = jnp.sum(c2.reshape(B // 8, 8, L), axis=2)
        c = c3.reshape(B, 1)
        mean = sum_ref[...] / jnp.maximum(c, 1.0)
        acc = lax.dot_general(mean, wl_ref[...], (((1,), (1,)), ((), ())),
                              precision=lax.Precision.HIGHEST,
                              preferred_element_type=jnp.float32)
        acc = acc + lax.dot_general(xt_ref[...], wr_ref[...],
                                    (((1,), (1,)), ((), ())),
                                    precision=lax.Precision.HIGHEST,
                                    preferred_element_type=jnp.float32)
        acc = acc + b_ref[...]
        o_ref[...] = jnp.maximum(acc, 0.0) if relu else acc

    return pl.pallas_call(
        body,
        grid=(N // B,),
        in_specs=[
            pl.BlockSpec((B, D), lambda i: (i, 0)),
            pl.BlockSpec((NS, B // 8, D), lambda i: (0, i, 0)),
            pl.BlockSpec((B, D), lambda i: (i, 0)),
            pl.BlockSpec((D, D), lambda i: (0, 0)),
            pl.BlockSpec((1, D), lambda i: (0, 0)),
            pl.BlockSpec((D, D), lambda i: (0, 0)),
        ],
        out_specs=pl.BlockSpec((B, D), lambda i: (i, 0)),
        out_shape=jax.ShapeDtypeStruct((N, D), jnp.float32),
    )(sums, cnt, x_tgt, W_l, b_l, W_r)


def kernel(x, src0, dst0, src1, dst1, W_l0, b_l0, W_r0, W_l1, b_l1, W_r1):
    sums0, cnt0 = _sc_segment_sum(x, src0, dst0, n_seg=N1, W=1408)
    h = _tc_linear(sums0, cnt0, x, W_l0, b_l0, W_r0, relu=True)
    sums1, cnt1 = _sc_segment_sum(h, src1, dst1, n_seg=N2, W=1536)
    h2 = _tc_linear(sums1, cnt1, h, W_l1, b_l1, W_r1, relu=False)
    third = N2 // 3
    return (h2[:third], h2[third:2 * third], h2[2 * third:])


# speedup vs baseline: 1.0819x; 1.0819x over previous
import functools

import jax
import jax.numpy as jnp
from jax import lax
from jax.experimental import pallas as pl
from jax.experimental.pallas import tpu as pltpu
from jax.experimental.pallas import tpu_sc as plsc

N0, N1, N2 = 540672, 33792, 3072
D = 128

NC, NS, L = 2, 16, 16
SUB = 128
SUPER = 1024
NSUB = SUPER // SUB

ZR = 8


def _sc_segment_sum(table, src, dst, n_seg, W):
    K = n_seg // W
    P = K // NC
    E = src.shape[0]
    Epad = E + SUPER
    src = jnp.pad(src.astype(jnp.int32), (0, Epad - E))
    dst = jnp.pad(dst.astype(jnp.int32), (0, Epad - E))
    b = jnp.arange(K + 1, dtype=jnp.int32) * W
    bounds = jnp.searchsorted(dst[:E], b, side="left").astype(jnp.int32)
    bounds = jnp.pad(bounds, (0, 32 - (K + 1)), mode="edge")

    R = W // NS
    CW = W // 8
    assert R % ZR == 0 and R % 8 == 0 and W % 8 == 0 and K + 1 <= 32
    assert CW % 8 == 0

    def body(table, src, dst, bounds, sums_hbm, cnt_hbm,
             sums, sidx, draw, didx, rows, cloc, zrow, bvec,
             semg, sems, semz):
        c = lax.axis_index("c")
        s = lax.axis_index("s")
        lanes = lax.iota(jnp.int32, L)
        one16 = jnp.ones((L,), jnp.float32)
        zero16 = jnp.zeros((L,), jnp.float32)

        for j in range(ZR):
            for qq in range(D // L):
                zrow[j, pl.ds(qq * L, L)] = zero16

        pltpu.sync_copy(bounds, bvec)
        bv0 = bvec[pl.ds(0, 16)]
        bv1 = bvec[pl.ds(16, 16)]

        def bnd(i):
            return bv0[i] if i < 16 else bv1[i - 16]

        for p in range(P):
            k = c * P + p
            lo = jnp.where(c == 0, bnd(p), bnd(P + p))
            hi = jnp.where(c == 0, bnd(p + 1), bnd(P + p + 1))
            segbase = k * W

            zcps = []
            for z in range(R // ZR):
                zcps.append(pltpu.async_copy(
                    zrow, sums.at[pl.ds(s * R + z * ZR, ZR)], semz))

            def zloc(r, carry2):
                for qq in range(D // L):
                    cloc[r, pl.ds(qq * L, L)] = zero16
                return carry2

            lax.fori_loop(0, CW + 8, zloc, 0)

            @pl.when(s == 0)
            def _():
                pltpu.async_copy(zrow.at[pl.ds(0, 8)],
                                 sums.at[pl.ds(W, 8)], semz).wait()

            for cp in zcps:
                cp.wait()
            plsc.subcore_barrier()

            q = (hi - lo + NS - 1) // NS
            t_lo = jnp.minimum(lo + s * q, hi)
            t_hi = jnp.minimum(t_lo + q, hi)
            s8 = t_lo - lax.rem(t_lo, 8)
            nsc = (t_hi - s8 + SUPER - 1) // SUPER

            def fire_gather(j):
                return pltpu.async_copy(
                    table.at[sidx.at[pl.ds(j * SUB, SUB)]],
                    rows.at[lax.rem(j, 2)], semg)

            def fire_scatters(j):
                pltpu.async_copy(rows.at[lax.rem(j, 2)],
                                 sums.at[didx.at[j]], sems, add=True)

            def wait_gather(j):
                pltpu.make_async_copy(
                    table.at[sidx.at[pl.ds(j * SUB, SUB)]],
                    rows.at[lax.rem(j, 2)], semg).wait()

            def wait_scatters(j):
                pltpu.make_async_copy(rows.at[lax.rem(j, 2)],
                                      sums.at[didx.at[j]], sems).wait()

            def superchunk(sch, carry):
                sc8 = pl.multiple_of(s8 + sch * SUPER, 8)
                pltpu.sync_copy(src.at[pl.ds(sc8, SUPER)], sidx)
                pltpu.sync_copy(dst.at[pl.ds(sc8, SUPER)], draw)
                rem = t_hi - sc8
                m = jnp.minimum((rem + SUB - 1) // SUB, NSUB)
                fire_gather(0)

                def prep(i, carry2):
                    dv = draw[pl.ds(i * L, L)]
                    g = sc8 + i * L + lanes
                    valid = (g >= t_lo) & (g < t_hi)
                    lidx = jnp.where(valid, dv - segbase, W)
                    didx[i // 8, pl.ds(lax.rem(i, 8) * L, L)] = lidx
                    plsc.addupdate_scatter(
                        cloc,
                        [lax.shift_right_logical(lidx, 3),
                         lax.shift_left(jnp.bitwise_and(lidx, 7), 4) + lanes],
                        one16)
                    return carry2

                lax.fori_loop(0, SUPER // L, prep, 0)

                def inner(j, carry2):
                    @pl.when(j >= 1)
                    def _():
                        wait_scatters(j - 1)

                    @pl.when(j + 1 < m)
                    def _():
                        fire_gather(j + 1)

                    wait_gather(j)
                    fire_scatters(j)
                    return carry2

                lax.fori_loop(0, m, inner, 0)
                wait_scatters(m - 1)
                return carry

            lax.fori_loop(0, nsc, superchunk, 0)
            plsc.subcore_barrier()

            cpo = [pltpu.async_copy(sums.at[pl.ds(s * R, R)],
                                    sums_hbm.at[pl.ds(segbase + s * R, R)],
                                    semz),
                   pltpu.async_copy(cloc.at[pl.ds(0, CW)],
                                    cnt_hbm.at[s, pl.ds(k * CW, CW)],
                                    semz)]
            for cp in cpo:
                cp.wait()
            plsc.subcore_barrier()

    mesh = plsc.VectorSubcoreMesh(core_axis_name="c", subcore_axis_name="s",
                                  num_cores=NC, num_subcores=NS)
    f = pl.kernel(
        body,
        out_type=(jax.ShapeDtypeStruct((n_seg, D), jnp.float32),
                  jax.ShapeDtypeStruct((NS, n_seg // 8, D), jnp.float32)),
        mesh=mesh,
        compiler_params=pltpu.CompilerParams(needs_layout_passes=False),
        scratch_types=[
            pltpu.VMEM_SHARED((W + 8, D), jnp.float32),
            pltpu.VMEM((SUPER,), jnp.int32),
            pltpu.VMEM((SUPER,), jnp.int32),
            pltpu.VMEM((NSUB, SUB), jnp.int32),
            pltpu.VMEM((2, SUB, D), jnp.float32),
            pltpu.VMEM((CW + 8, D), jnp.float32),
            pltpu.VMEM((ZR, D), jnp.float32),
            pltpu.VMEM((32,), jnp.int32),
            pltpu.SemaphoreType.DMA,
            pltpu.SemaphoreType.DMA,
            pltpu.SemaphoreType.DMA,
        ],
    )
    return f(table, src, dst, bounds)


def _tc_linear(sums, cnt, x_tgt, W_l, b_l, W_r, relu):
    N = sums.shape[0]
    B = 1024
    b_l = b_l.reshape(1, D)

    def body(sum_ref, cnt_ref, xt_ref, wl_ref, b_ref, wr_ref, o_ref):
        c2 = jnp.sum(cnt_ref[...], axis=0)
        c3 = jnp.sum(c2.reshape(B // 8, 8, L), axis=2)
        c = c3.reshape(B, 1)
        mean = sum_ref[...] / jnp.maximum(c, 1.0)
        acc = lax.dot_general(mean, wl_ref[...], (((1,), (1,)), ((), ())),
                              precision=lax.Precision.HIGHEST,
                              preferred_element_type=jnp.float32)
        acc = acc + lax.dot_general(xt_ref[...], wr_ref[...],
                                    (((1,), (1,)), ((), ())),
                                    precision=lax.Precision.HIGHEST,
                                    preferred_element_type=jnp.float32)
        acc = acc + b_ref[...]
        o_ref[...] = jnp.maximum(acc, 0.0) if relu else acc

    return pl.pallas_call(
        body,
        grid=(N // B,),
        in_specs=[
            pl.BlockSpec((B, D), lambda i: (i, 0)),
            pl.BlockSpec((NS, B // 8, D), lambda i: (0, i, 0)),
            pl.BlockSpec((B, D), lambda i: (i, 0)),
            pl.BlockSpec((D, D), lambda i: (0, 0)),
            pl.BlockSpec((1, D), lambda i: (0, 0)),
            pl.BlockSpec((D, D), lambda i: (0, 0)),
        ],
        out_specs=pl.BlockSpec((B, D), lambda i: (i, 0)),
        out_shape=jax.ShapeDtypeStruct((N, D), jnp.float32),
    )(sums, cnt, x_tgt, W_l, b_l, W_r)


def kernel(x, src0, dst0, src1, dst1, W_l0, b_l0, W_r0, W_l1, b_l1, W_r1):
    sums0, cnt0 = _sc_segment_sum(x, src0, dst0, n_seg=N1, W=2816)
    h = _tc_linear(sums0, cnt0, x, W_l0, b_l0, W_r0, relu=True)
    sums1, cnt1 = _sc_segment_sum(h, src1, dst1, n_seg=N2, W=1536)
    h2 = _tc_linear(sums1, cnt1, h, W_l1, b_l1, W_r1, relu=False)
    third = N2 // 3
    return (h2[:third], h2[third:2 * third], h2[2 * third:])


# speedup vs baseline: 1.1371x; 1.0510x over previous
import functools

import jax
import jax.numpy as jnp
from jax import lax
from jax.experimental import pallas as pl
from jax.experimental.pallas import tpu as pltpu
from jax.experimental.pallas import tpu_sc as plsc

N0, N1, N2 = 540672, 33792, 3072
D = 128

NC, NS, L = 2, 16, 16
SUB = 128
SUPER = 1024
NSUB = SUPER // SUB

ZR = 8


def _sc_segment_sum(table, src, dst, n_seg, W):
    K = n_seg // W
    P = K // NC
    E = src.shape[0]
    Epad = E + SUPER
    src = jnp.pad(src.astype(jnp.int32), (0, Epad - E))
    dst = jnp.pad(dst.astype(jnp.int32), (0, Epad - E))
    b = jnp.arange(K + 1, dtype=jnp.int32) * W
    bounds = jnp.searchsorted(dst[:E], b, side="left").astype(jnp.int32)
    bounds = jnp.pad(bounds, (0, 32 - (K + 1)), mode="edge")

    R = W // NS
    CW = W // 8
    assert R % ZR == 0 and R % 8 == 0 and W % 8 == 0 and K + 1 <= 32
    assert CW % 8 == 0

    def body(table, src, dst, bounds, sums_hbm, cnt_hbm,
             sums, sidx, draw, didx, rows, cloc, zrow, bvec,
             semg, sems, semz, semi):
        c = lax.axis_index("c")
        s = lax.axis_index("s")
        lanes = lax.iota(jnp.int32, L)
        one16 = jnp.ones((L,), jnp.float32)
        zero16 = jnp.zeros((L,), jnp.float32)

        for j in range(ZR):
            for qq in range(D // L):
                zrow[j, pl.ds(qq * L, L)] = zero16

        pltpu.sync_copy(bounds, bvec)
        bv0 = bvec[pl.ds(0, 16)]
        bv1 = bvec[pl.ds(16, 16)]

        def bnd(i):
            return bv0[i] if i < 16 else bv1[i - 16]

        for p in range(P):
            k = c * P + p
            lo = jnp.where(c == 0, bnd(p), bnd(P + p))
            hi = jnp.where(c == 0, bnd(p + 1), bnd(P + p + 1))
            segbase = k * W

            zcps = []
            for z in range(R // ZR):
                zcps.append(pltpu.async_copy(
                    zrow, sums.at[pl.ds(s * R + z * ZR, ZR)], semz))

            def zloc(r, carry2):
                for qq in range(D // L):
                    cloc[r, pl.ds(qq * L, L)] = zero16
                return carry2

            lax.fori_loop(0, CW + 8, zloc, 0)

            @pl.when(s == 0)
            def _():
                pltpu.async_copy(zrow.at[pl.ds(0, 8)],
                                 sums.at[pl.ds(W, 8)], semz).wait()

            for cp in zcps:
                cp.wait()
            plsc.subcore_barrier()

            q = (hi - lo + NS - 1) // NS
            t_lo = jnp.minimum(lo + s * q, hi)
            t_hi = jnp.minimum(t_lo + q, hi)
            s8 = t_lo - lax.rem(t_lo, 8)
            nsc = (t_hi - s8 + SUPER - 1) // SUPER

            def idx_copies(sch):
                off = pl.multiple_of(s8 + sch * SUPER, 8)
                hb = lax.rem(sch, 2) * SUPER
                return [(src.at[pl.ds(off, SUPER)],
                         sidx.at[pl.ds(hb, SUPER)]),
                        (dst.at[pl.ds(off, SUPER)],
                         draw.at[pl.ds(hb, SUPER)])]

            def fire_idx(sch):
                for a, b2 in idx_copies(sch):
                    pltpu.async_copy(a, b2, semi)

            def wait_idx(sch):
                for a, b2 in idx_copies(sch):
                    pltpu.make_async_copy(a, b2, semi).wait()

            def fire_gather(bb, j):
                return pltpu.async_copy(
                    table.at[sidx.at[pl.ds(bb * SUPER + j * SUB, SUB)]],
                    rows.at[lax.rem(j, 2)], semg)

            def fire_scatters(j):
                pltpu.async_copy(rows.at[lax.rem(j, 2)],
                                 sums.at[didx.at[j]], sems, add=True)

            def wait_gather(bb, j):
                pltpu.make_async_copy(
                    table.at[sidx.at[pl.ds(bb * SUPER + j * SUB, SUB)]],
                    rows.at[lax.rem(j, 2)], semg).wait()

            def wait_scatters(j):
                pltpu.make_async_copy(rows.at[lax.rem(j, 2)],
                                      sums.at[didx.at[j]], sems).wait()

            def superchunk(sch, carry):
                bb = lax.rem(sch, 2)
                sc8 = pl.multiple_of(s8 + sch * SUPER, 8)
                wait_idx(sch)

                @pl.when(sch + 1 < nsc)
                def _():
                    fire_idx(sch + 1)

                rem = t_hi - sc8
                m = jnp.minimum((rem + SUB - 1) // SUB, NSUB)
                fire_gather(bb, 0)

                def prep(i, carry2):
                    dv = draw[pl.ds(bb * SUPER + i * L, L)]
                    g = sc8 + i * L + lanes
                    valid = (g >= t_lo) & (g < t_hi)
                    lidx = jnp.where(valid, dv - segbase, W)
                    didx[i // 8, pl.ds(lax.rem(i, 8) * L, L)] = lidx
                    plsc.addupdate_scatter(
                        cloc,
                        [lax.shift_right_logical(lidx, 3),
                         lax.shift_left(jnp.bitwise_and(lidx, 7), 4) + lanes],
                        one16)
                    return carry2

                lax.fori_loop(0, SUPER // L, prep, 0)

                def inner(j, carry2):
                    @pl.when(j >= 1)
                    def _():
                        wait_scatters(j - 1)

                    @pl.when(j + 1 < m)
                    def _():
                        fire_gather(bb, j + 1)

                    wait_gather(bb, j)
                    fire_scatters(j)
                    return carry2

                lax.fori_loop(0, m, inner, 0)
                wait_scatters(m - 1)
                return carry

            @pl.when(nsc >= 1)
            def _():
                fire_idx(0)

            lax.fori_loop(0, nsc, superchunk, 0)
            plsc.subcore_barrier()

            cpo = [pltpu.async_copy(sums.at[pl.ds(s * R, R)],
                                    sums_hbm.at[pl.ds(segbase + s * R, R)],
                                    semz),
                   pltpu.async_copy(cloc.at[pl.ds(0, CW)],
                                    cnt_hbm.at[s, pl.ds(k * CW, CW)],
                                    semz)]
            for cp in cpo:
                cp.wait()
            plsc.subcore_barrier()

    mesh = plsc.VectorSubcoreMesh(core_axis_name="c", subcore_axis_name="s",
                                  num_cores=NC, num_subcores=NS)
    f = pl.kernel(
        body,
        out_type=(jax.ShapeDtypeStruct((n_seg, D), jnp.float32),
                  jax.ShapeDtypeStruct((NS, n_seg // 8, D), jnp.float32)),
        mesh=mesh,
        compiler_params=pltpu.CompilerParams(needs_layout_passes=False),
        scratch_types=[
            pltpu.VMEM_SHARED((W + 8, D), jnp.float32),
            pltpu.VMEM((2 * SUPER,), jnp.int32),
            pltpu.VMEM((2 * SUPER,), jnp.int32),
            pltpu.VMEM((NSUB, SUB), jnp.int32),
            pltpu.VMEM((2, SUB, D), jnp.float32),
            pltpu.VMEM((CW + 8, D), jnp.float32),
            pltpu.VMEM((ZR, D), jnp.float32),
            pltpu.VMEM((32,), jnp.int32),
            pltpu.SemaphoreType.DMA,
            pltpu.SemaphoreType.DMA,
            pltpu.SemaphoreType.DMA,
            pltpu.SemaphoreType.DMA,
        ],
    )
    return f(table, src, dst, bounds)


def _tc_linear(sums, cnt, x_tgt, W_l, b_l, W_r, relu):
    N = sums.shape[0]
    B = 1024
    b_l = b_l.reshape(1, D)

    def body(sum_ref, cnt_ref, xt_ref, wl_ref, b_ref, wr_ref, o_ref):
        c2 = jnp.sum(cnt_ref[...], axis=0)
        c3 = jnp.sum(c2.reshape(B // 8, 8, L), axis=2)
        c = c3.reshape(B, 1)
        mean = sum_ref[...] / jnp.maximum(c, 1.0)
        acc = lax.dot_general(mean, wl_ref[...], (((1,), (1,)), ((), ())),
                              precision=lax.Precision.HIGHEST,
                              preferred_element_type=jnp.float32)
        acc = acc + lax.dot_general(xt_ref[...], wr_ref[...],
                                    (((1,), (1,)), ((), ())),
                                    precision=lax.Precision.HIGHEST,
                                    preferred_element_type=jnp.float32)
        acc = acc + b_ref[...]
        o_ref[...] = jnp.maximum(acc, 0.0) if relu else acc

    return pl.pallas_call(
        body,
        grid=(N // B,),
        in_specs=[
            pl.BlockSpec((B, D), lambda i: (i, 0)),
            pl.BlockSpec((NS, B // 8, D), lambda i: (0, i, 0)),
            pl.BlockSpec((B, D), lambda i: (i, 0)),
            pl.BlockSpec((D, D), lambda i: (0, 0)),
            pl.BlockSpec((1, D), lambda i: (0, 0)),
            pl.BlockSpec((D, D), lambda i: (0, 0)),
        ],
        out_specs=pl.BlockSpec((B, D), lambda i: (i, 0)),
        out_shape=jax.ShapeDtypeStruct((N, D), jnp.float32),
    )(sums, cnt, x_tgt, W_l, b_l, W_r)


def kernel(x, src0, dst0, src1, dst1, W_l0, b_l0, W_r0, W_l1, b_l1, W_r1):
    sums0, cnt0 = _sc_segment_sum(x, src0, dst0, n_seg=N1, W=2816)
    h = _tc_linear(sums0, cnt0, x, W_l0, b_l0, W_r0, relu=True)
    sums1, cnt1 = _sc_segment_sum(h, src1, dst1, n_seg=N2, W=1536)
    h2 = _tc_linear(sums1, cnt1, h, W_l1, b_l1, W_r1, relu=False)
    third = N2 // 3
    return (h2[:third], h2[third:2 * third], h2[2 * third:])


# speedup vs baseline: 1.1394x; 1.0021x over previous
import jax
import jax.numpy as jnp
from jax import lax
from jax.experimental import pallas as pl
from jax.experimental.pallas import tpu as pltpu
from jax.experimental.pallas import tpu_sc as plsc

N0, N1, N2 = 540672, 33792, 3072
D = 128

NC, NS, L = 2, 16, 16
SUB = 128
SUPER = 1024
NSUB = SUPER // SUB

ZR = 8


def _sc_segment_sum(table, src, dst, n_seg, W):
    K = n_seg // W
    P = K // NC
    E = src.shape[0]
    Epad = E + SUPER
    src = jnp.pad(src.astype(jnp.int32), (0, Epad - E))
    dst = jnp.pad(dst.astype(jnp.int32), (0, Epad - E))
    b = jnp.arange(K + 1, dtype=jnp.int32) * W
    bounds = jnp.searchsorted(dst[:E], b, side="left").astype(jnp.int32)
    bounds = jnp.pad(bounds, (0, 32 - (K + 1)), mode="edge")

    R = W // NS
    CW = W // 8
    assert R % ZR == 0 and R % 8 == 0 and W % 8 == 0 and K + 1 <= 32
    assert CW % 8 == 0

    def body(table, src, dst, bounds, sums_hbm, cnt_hbm,
             sums, sidx, draw, didx, rows, cloc, zrow, bvec,
             semg, sems, semz, semi):
        c = lax.axis_index("c")
        s = lax.axis_index("s")
        lanes = lax.iota(jnp.int32, L)
        one16 = jnp.ones((L,), jnp.float32)
        zero16 = jnp.zeros((L,), jnp.float32)

        for j in range(ZR):
            for qq in range(D // L):
                zrow[j, pl.ds(qq * L, L)] = zero16

        pltpu.sync_copy(bounds, bvec)
        bv0 = bvec[pl.ds(0, 16)]
        bv1 = bvec[pl.ds(16, 16)]

        def bnd(i):
            return bv0[i] if i < 16 else bv1[i - 16]

        for p in range(P):
            k = c * P + p
            lo = jnp.where(c == 0, bnd(p), bnd(P + p))
            hi = jnp.where(c == 0, bnd(p + 1), bnd(P + p + 1))
            segbase = k * W

            zcps = []
            for z in range(R // ZR):
                zcps.append(pltpu.async_copy(
                    zrow, sums.at[pl.ds(s * R + z * ZR, ZR)], semz))

            def zloc(r, carry2):
                for qq in range(D // L):
                    cloc[r, pl.ds(qq * L, L)] = zero16
                return carry2

            lax.fori_loop(0, CW + 8, zloc, 0)

            @pl.when(s == 0)
            def _():
                pltpu.async_copy(zrow.at[pl.ds(0, 8)],
                                 sums.at[pl.ds(W, 8)], semz).wait()

            for cp in zcps:
                cp.wait()
            plsc.subcore_barrier()

            q = (hi - lo + NS - 1) // NS
            t_lo = jnp.minimum(lo + s * q, hi)
            t_hi = jnp.minimum(t_lo + q, hi)
            s8 = t_lo - lax.rem(t_lo, 8)
            nsc = (t_hi - s8 + SUPER - 1) // SUPER

            def idx_copies(sch):
                off = pl.multiple_of(s8 + sch * SUPER, 8)
                hb = lax.rem(sch, 2) * SUPER
                return [(src.at[pl.ds(off, SUPER)],
                         sidx.at[pl.ds(hb, SUPER)]),
                        (dst.at[pl.ds(off, SUPER)],
                         draw.at[pl.ds(hb, SUPER)])]

            def fire_idx(sch):
                for a, b2 in idx_copies(sch):
                    pltpu.async_copy(a, b2, semi)

            def wait_idx(sch):
                for a, b2 in idx_copies(sch):
                    pltpu.make_async_copy(a, b2, semi).wait()

            def fire_gather(bb, j):
                return pltpu.async_copy(
                    table.at[sidx.at[pl.ds(bb * SUPER + j * SUB, SUB)]],
                    rows.at[lax.rem(j, 2)], semg)

            def fire_scatters(j):
                pltpu.async_copy(rows.at[lax.rem(j, 2)],
                                 sums.at[didx.at[j]], sems, add=True)

            def wait_gather(bb, j):
                pltpu.make_async_copy(
                    table.at[sidx.at[pl.ds(bb * SUPER + j * SUB, SUB)]],
                    rows.at[lax.rem(j, 2)], semg).wait()

            def wait_scatters(j):
                pltpu.make_async_copy(rows.at[lax.rem(j, 2)],
                                      sums.at[didx.at[j]], sems).wait()

            def superchunk(sch, carry):
                bb = lax.rem(sch, 2)
                sc8 = pl.multiple_of(s8 + sch * SUPER, 8)
                wait_idx(sch)

                @pl.when(sch + 1 < nsc)
                def _():
                    fire_idx(sch + 1)

                rem = t_hi - sc8
                m = jnp.minimum((rem + SUB - 1) // SUB, NSUB)
                fire_gather(bb, 0)

                def prep(i, carry2):
                    dv = draw[pl.ds(bb * SUPER + i * L, L)]
                    g = sc8 + i * L + lanes
                    valid = (g >= t_lo) & (g < t_hi)
                    lidx = jnp.where(valid, dv - segbase, W)
                    didx[i // 8, pl.ds(lax.rem(i, 8) * L, L)] = lidx
                    plsc.addupdate_scatter(
                        cloc,
                        [lax.shift_right_logical(lidx, 3),
                         lax.shift_left(jnp.bitwise_and(lidx, 7), 4) + lanes],
                        one16)
                    return carry2

                lax.fori_loop(0, SUPER // L, prep, 0)

                def inner(j, carry2):
                    @pl.when(j >= 1)
                    def _():
                        wait_scatters(j - 1)

                    @pl.when(j + 1 < m)
                    def _():
                        fire_gather(bb, j + 1)

                    wait_gather(bb, j)
                    fire_scatters(j)
                    return carry2

                lax.fori_loop(0, m, inner, 0)
                wait_scatters(m - 1)
                return carry

            @pl.when(nsc >= 1)
            def _():
                fire_idx(0)

            lax.fori_loop(0, nsc, superchunk, 0)
            plsc.subcore_barrier()

            cpo = [pltpu.async_copy(sums.at[pl.ds(s * R, R)],
                                    sums_hbm.at[pl.ds(segbase + s * R, R)],
                                    semz),
                   pltpu.async_copy(cloc.at[pl.ds(0, CW)],
                                    cnt_hbm.at[s, pl.ds(k * CW, CW)],
                                    semz)]
            for cp in cpo:
                cp.wait()
            plsc.subcore_barrier()

    mesh = plsc.VectorSubcoreMesh(core_axis_name="c", subcore_axis_name="s",
                                  num_cores=NC, num_subcores=NS)
    f = pl.kernel(
        body,
        out_type=(jax.ShapeDtypeStruct((n_seg, D), jnp.float32),
                  jax.ShapeDtypeStruct((NS, n_seg // 8, D), jnp.float32)),
        mesh=mesh,
        compiler_params=pltpu.CompilerParams(needs_layout_passes=False),
        scratch_types=[
            pltpu.VMEM_SHARED((W + 8, D), jnp.float32),
            pltpu.VMEM((2 * SUPER,), jnp.int32),
            pltpu.VMEM((2 * SUPER,), jnp.int32),
            pltpu.VMEM((NSUB, SUB), jnp.int32),
            pltpu.VMEM((2, SUB, D), jnp.float32),
            pltpu.VMEM((CW + 8, D), jnp.float32),
            pltpu.VMEM((ZR, D), jnp.float32),
            pltpu.VMEM((32,), jnp.int32),
            pltpu.SemaphoreType.DMA,
            pltpu.SemaphoreType.DMA,
            pltpu.SemaphoreType.DMA,
            pltpu.SemaphoreType.DMA,
        ],
    )
    return f(table, src, dst, bounds)


def _tc_linear(sums, cnt, x_tgt, W_l, b_l, W_r, relu):
    N = sums.shape[0]
    B = 1024
    b_l = b_l.reshape(1, D)

    def body(sum_ref, cnt_ref, xt_ref, wl_ref, b_ref, wr_ref, o_ref):
        c2 = jnp.sum(cnt_ref[...], axis=0)
        c3 = jnp.sum(c2.reshape(B // 8, 8, L), axis=2)
        c = c3.reshape(B, 1)
        mean = sum_ref[...] / jnp.maximum(c, 1.0)
        acc = lax.dot_general(mean, wl_ref[...], (((1,), (1,)), ((), ())),
                              precision=lax.Precision.HIGHEST,
                              preferred_element_type=jnp.float32)
        acc = acc + lax.dot_general(xt_ref[...], wr_ref[...],
                                    (((1,), (1,)), ((), ())),
                                    precision=lax.Precision.HIGHEST,
                                    preferred_element_type=jnp.float32)
        acc = acc + b_ref[...]
        o_ref[...] = jnp.maximum(acc, 0.0) if relu else acc

    return pl.pallas_call(
        body,
        grid=(N // B,),
        in_specs=[
            pl.BlockSpec((B, D), lambda i: (i, 0)),
            pl.BlockSpec((NS, B // 8, D), lambda i: (0, i, 0)),
            pl.BlockSpec((B, D), lambda i: (i, 0)),
            pl.BlockSpec((D, D), lambda i: (0, 0)),
            pl.BlockSpec((1, D), lambda i: (0, 0)),
            pl.BlockSpec((D, D), lambda i: (0, 0)),
        ],
        out_specs=pl.BlockSpec((B, D), lambda i: (i, 0)),
        out_shape=jax.ShapeDtypeStruct((N, D), jnp.float32),
    )(sums, cnt, x_tgt, W_l, b_l, W_r)


def kernel(x, src0, dst0, src1, dst1, W_l0, b_l0, W_r0, W_l1, b_l1, W_r1):
    sums0, cnt0 = _sc_segment_sum(x, src0, dst0, n_seg=N1, W=2816)
    h = _tc_linear(sums0, cnt0, x, W_l0, b_l0, W_r0, relu=True)
    sums1, cnt1 = _sc_segment_sum(h, src1, dst1, n_seg=N2, W=1536)
    h2 = _tc_linear(sums1, cnt1, h, W_l1, b_l1, W_r1, relu=False)
    third = N2 // 3
    return (h2[:third], h2[third:2 * third], h2[2 * third:])
